# Initial kernel scaffold; baseline (speedup 1.0000x reference)
import jax
import jax.numpy as jnp
from jax.experimental import pallas as pl


def kernel(decodings, variables, dec_sem_logits, gumbel_noise, target_types, spans):
    raise NotImplementedError("write your pallas kernel here")


# speedup vs baseline: 7.6213x; 7.6213x over previous
import jax
import jax.numpy as jnp
from jax.experimental import pallas as pl
from jax.experimental.pallas import tpu as pltpu

B, N, M, V = 128, 8, 64, 512
K = 8
T = 30
PAD = 0
NEG_INF = -1e30


def _kernel(spans_ref, tt_ref, logits_ref, gumbel_ref, dec_ref, out_ref):
    b = pl.program_id(0)
    span_b = spans_ref[b]
    tt_b = tt_ref[b]

    logits = logits_ref[0, 0]
    g = gumbel_ref[0]
    col = jax.lax.broadcasted_iota(jnp.int32, (K, N + 1), 1)
    masked = jnp.where(col <= span_b, logits, NEG_INF)
    shifted = masked - jnp.max(masked, axis=1, keepdims=True)
    logp = shifted - jnp.log(jnp.sum(jnp.exp(shifted), axis=1, keepdims=True))
    z = logp + g
    ez = jnp.exp(z - jnp.max(z, axis=1, keepdims=True))
    soft = ez / jnp.sum(ez, axis=1, keepdims=True)
    sel = jnp.argmax(soft, axis=1).reshape(K, 1).astype(jnp.int32)
    krow = jax.lax.broadcasted_iota(jnp.int32, (K, 1), 0)
    sel = jnp.where(tt_b == 20, jnp.where(krow == 0, 1, 0), sel)

    dec = dec_ref[0]
    rowmax = jnp.max(dec, axis=2)
    col0 = dec[:, :, 0]
    nidx = jax.lax.broadcasted_iota(jnp.int32, (K, N), 1)
    onehot = (nidx == (sel - 1)).astype(jnp.float32)
    rowmax_sel = jax.lax.dot(onehot, rowmax,
                             precision=jax.lax.Precision.HIGHEST)
    col0_sel = jax.lax.dot(onehot, col0,
                           precision=jax.lax.Precision.HIGHEST)
    midx = jax.lax.broadcasted_iota(jnp.int32, (K, M), 1)
    nz = col0_sel < rowmax_sel
    lens = jnp.max(jnp.where(nz, midx + 1, 0), axis=1).reshape(K, 1)

    kk_r = jax.lax.broadcasted_iota(jnp.int32, (K, K), 0)
    kk_c = jax.lax.broadcasted_iota(jnp.int32, (K, K), 1)
    lens_row = lens.reshape(1, K)
    excl = jnp.sum(jnp.where(kk_c < kk_r, jnp.broadcast_to(lens_row, (K, K)), 0),
                   axis=1).reshape(K, 1)
    idx = jnp.minimum(excl, M)
    olen = jnp.minimum(lens, M - idx)

    j = jax.lax.broadcasted_iota(jnp.int32, (K, M), 1)
    in_seg = (j >= idx) & (j < idx + olen)
    srcval = (sel - 1) * M + (j - idx)
    src = jnp.sum(jnp.where(in_seg, srcval, 0), axis=0).reshape(M, 1)
    covered = jnp.sum(in_seg.astype(jnp.int32), axis=0).reshape(M, 1) > 0
    src = jnp.where(covered, src, -1)

    i_flat = jax.lax.broadcasted_iota(jnp.int32, (M, N * M), 1)
    P = (i_flat == src).astype(jnp.float32)
    dec_flat = dec.reshape(N * M, V)
    out_ref[0] = jax.lax.dot(P, dec_flat,
                             precision=jax.lax.Precision.HIGHEST,
                             preferred_element_type=jnp.float32)


def kernel(decodings, variables, dec_sem_logits, gumbel_noise, target_types, spans):
    del variables
    spans = spans.astype(jnp.int32)
    target_types = target_types.astype(jnp.int32)

    grid_spec = pltpu.PrefetchScalarGridSpec(
        num_scalar_prefetch=2,
        grid=(B,),
        in_specs=[
            pl.BlockSpec((1, 1, K, N + 1),
                         lambda b, spans_ref, tt_ref: (spans_ref[b] - 2,
                                                       tt_ref[b] - 9, 0, 0)),
            pl.BlockSpec((1, K, N + 1), lambda b, *_: (b, 0, 0)),
            pl.BlockSpec((1, N, M, V), lambda b, *_: (b, 0, 0, 0)),
        ],
        out_specs=pl.BlockSpec((1, M, V), lambda b, *_: (b, 0, 0)),
    )
    return pl.pallas_call(
        _kernel,
        out_shape=jax.ShapeDtypeStruct((B, M, V), jnp.float32),
        grid_spec=grid_spec,
        compiler_params=pltpu.CompilerParams(
            dimension_semantics=("parallel",),
        ),
        name="tree_lstm_template",
    )(spans, target_types, dec_sem_logits, gumbel_noise, decodings)


# speedup vs baseline: 10.7009x; 1.4041x over previous
import jax
import jax.numpy as jnp
from jax.experimental import pallas as pl
from jax.experimental.pallas import tpu as pltpu

B, N, M, V = 128, 8, 64, 512
K = 8
T = 30
PAD = 0
NEG_INF = -1e30


def _kernel(spans_ref, tt_ref, logits_ref, gumbel_ref, dec_ref, out_ref):
    b = pl.program_id(0)
    span_b = spans_ref[b]
    tt_b = tt_ref[b]

    logits = logits_ref[0, 0]
    g = gumbel_ref[0]
    col = jax.lax.broadcasted_iota(jnp.int32, (K, N + 1), 1)
    masked = jnp.where(col <= span_b, logits, NEG_INF)
    shifted = masked - jnp.max(masked, axis=1, keepdims=True)
    logp = shifted - jnp.log(jnp.sum(jnp.exp(shifted), axis=1, keepdims=True))
    z = logp + g
    ez = jnp.exp(z - jnp.max(z, axis=1, keepdims=True))
    soft = ez / jnp.sum(ez, axis=1, keepdims=True)
    sel = jnp.argmax(soft, axis=1).reshape(K, 1).astype(jnp.int32)
    krow = jax.lax.broadcasted_iota(jnp.int32, (K, 1), 0)
    sel = jnp.where(tt_b == 20, jnp.where(krow == 0, 1, 0), sel)

    dec = dec_ref[0]
    rowmax = jnp.max(dec, axis=2)
    col0 = dec[:, :, 0]
    nsel = jnp.broadcast_to(jnp.maximum(sel - 1, 0), (K, M))
    rowmax_sel = jnp.take_along_axis(rowmax, nsel, axis=0)
    col0_sel = jnp.take_along_axis(col0, nsel, axis=0)
    midx = jax.lax.broadcasted_iota(jnp.int32, (K, M), 1)
    nz = (col0_sel < rowmax_sel) & (sel > 0)
    lens = jnp.max(jnp.where(nz, midx + 1, 0), axis=1).reshape(K, 1)

    kk_r = jax.lax.broadcasted_iota(jnp.int32, (K, K), 0)
    kk_c = jax.lax.broadcasted_iota(jnp.int32, (K, K), 1)
    lens_row = lens.reshape(1, K)
    excl = jnp.sum(jnp.where(kk_c < kk_r, jnp.broadcast_to(lens_row, (K, K)), 0),
                   axis=1).reshape(K, 1)
    idx = jnp.minimum(excl, M)
    olen = jnp.minimum(lens, M - idx)

    j = jax.lax.broadcasted_iota(jnp.int32, (K, M), 1)
    in_seg = (j >= idx) & (j < idx + olen)
    srcval = (sel - 1) * M + (j - idx)
    src = jnp.sum(jnp.where(in_seg, srcval, 0), axis=0).reshape(M, 1)
    covered = jnp.sum(in_seg.astype(jnp.int32), axis=0).reshape(M, 1) > 0
    src = jnp.where(covered, src, -1)

    i_flat = jax.lax.broadcasted_iota(jnp.int32, (M, N * M), 1)
    P = (i_flat == src).astype(jnp.float32)
    dec_flat = dec.reshape(N * M, V)
    out_ref[0] = jax.lax.dot(P, dec_flat,
                             preferred_element_type=jnp.float32)


def kernel(decodings, variables, dec_sem_logits, gumbel_noise, target_types, spans):
    del variables
    spans = spans.astype(jnp.int32)
    target_types = target_types.astype(jnp.int32)

    grid_spec = pltpu.PrefetchScalarGridSpec(
        num_scalar_prefetch=2,
        grid=(B,),
        in_specs=[
            pl.BlockSpec((1, 1, K, N + 1),
                         lambda b, spans_ref, tt_ref: (spans_ref[b] - 2,
                                                       tt_ref[b] - 9, 0, 0)),
            pl.BlockSpec((1, K, N + 1), lambda b, *_: (b, 0, 0)),
            pl.BlockSpec((1, N, M, V), lambda b, *_: (b, 0, 0, 0)),
        ],
        out_specs=pl.BlockSpec((1, M, V), lambda b, *_: (b, 0, 0)),
    )
    return pl.pallas_call(
        _kernel,
        out_shape=jax.ShapeDtypeStruct((B, M, V), jnp.float32),
        grid_spec=grid_spec,
        compiler_params=pltpu.CompilerParams(
            dimension_semantics=("parallel",),
        ),
        name="tree_lstm_template",
    )(spans, target_types, dec_sem_logits, gumbel_noise, decodings)


# speedup vs baseline: 19.4436x; 1.8170x over previous
import jax
import jax.numpy as jnp
from jax.experimental import pallas as pl
from jax.experimental.pallas import tpu as pltpu

B, N, M, V = 128, 8, 64, 512
K = 8
T = 30
PAD = 0
NEG_INF = -1e30
G = 8


def _one_sample(logits, g, span_b, tt_b, dec):
    col = jax.lax.broadcasted_iota(jnp.int32, (K, N + 1), 1)
    masked = jnp.where(col <= span_b, logits, NEG_INF)
    shifted = masked - jnp.max(masked, axis=1, keepdims=True)
    logp = shifted - jnp.log(jnp.sum(jnp.exp(shifted), axis=1, keepdims=True))
    z = logp + g
    ez = jnp.exp(z - jnp.max(z, axis=1, keepdims=True))
    soft = ez / jnp.sum(ez, axis=1, keepdims=True)
    sel = jnp.argmax(soft, axis=1).reshape(K, 1).astype(jnp.int32)
    krow = jax.lax.broadcasted_iota(jnp.int32, (K, 1), 0)
    sel = jnp.where(tt_b == 20, jnp.where(krow == 0, 1, 0), sel)

    rowmax = jnp.max(dec, axis=2)
    col0 = dec[:, :, 0]
    nsel = jnp.broadcast_to(jnp.maximum(sel - 1, 0), (K, M))
    rowmax_sel = jnp.take_along_axis(rowmax, nsel, axis=0)
    col0_sel = jnp.take_along_axis(col0, nsel, axis=0)
    midx = jax.lax.broadcasted_iota(jnp.int32, (K, M), 1)
    nz = (col0_sel < rowmax_sel) & (sel > 0)
    lens = jnp.max(jnp.where(nz, midx + 1, 0), axis=1).reshape(K, 1)

    kk_r = jax.lax.broadcasted_iota(jnp.int32, (K, K), 0)
    kk_c = jax.lax.broadcasted_iota(jnp.int32, (K, K), 1)
    lens_row = lens.reshape(1, K)
    excl = jnp.sum(jnp.where(kk_c < kk_r, jnp.broadcast_to(lens_row, (K, K)), 0),
                   axis=1).reshape(K, 1)
    idx = jnp.minimum(excl, M)
    olen = jnp.minimum(lens, M - idx)

    j = jax.lax.broadcasted_iota(jnp.int32, (K, M), 1)
    in_seg = (j >= idx) & (j < idx + olen)
    srcval = (sel - 1) * M + (j - idx)
    src = jnp.sum(jnp.where(in_seg, srcval, 0), axis=0).reshape(M, 1)
    covered = jnp.sum(in_seg.astype(jnp.int32), axis=0).reshape(M, 1) > 0
    src = jnp.where(covered, src, -1)

    i_flat = jax.lax.broadcasted_iota(jnp.int32, (M, N * M), 1)
    P = (i_flat == src).astype(jnp.float32)
    dec_flat = dec.reshape(N * M, V)
    return jax.lax.dot(P, dec_flat, preferred_element_type=jnp.float32)


def _kernel(spans_ref, tt_ref, logits_ref, gumbel_ref, dec_ref, out_ref):
    i = pl.program_id(0)
    for gi in range(G):
        b = i * G + gi
        span_b = spans_ref[b]
        tt_b = tt_ref[b]
        logits = logits_ref[span_b - 2, tt_b - 9]
        gum = gumbel_ref[gi]
        dec = dec_ref[gi]
        out_ref[gi] = _one_sample(logits, gum, span_b, tt_b, dec)


def kernel(decodings, variables, dec_sem_logits, gumbel_noise, target_types, spans):
    del variables
    spans = spans.astype(jnp.int32)
    target_types = target_types.astype(jnp.int32)

    grid_spec = pltpu.PrefetchScalarGridSpec(
        num_scalar_prefetch=2,
        grid=(B // G,),
        in_specs=[
            pl.BlockSpec((N - 1, T - 9, K, N + 1), lambda i, *_: (0, 0, 0, 0)),
            pl.BlockSpec((G, K, N + 1), lambda i, *_: (i, 0, 0)),
            pl.BlockSpec((G, N, M, V), lambda i, *_: (i, 0, 0, 0)),
        ],
        out_specs=pl.BlockSpec((G, M, V), lambda i, *_: (i, 0, 0)),
    )
    return pl.pallas_call(
        _kernel,
        out_shape=jax.ShapeDtypeStruct((B, M, V), jnp.float32),
        grid_spec=grid_spec,
        compiler_params=pltpu.CompilerParams(
            dimension_semantics=("parallel",),
            vmem_limit_bytes=100 * 1024 * 1024,
        ),
        name="tree_lstm_template",
    )(spans, target_types, dec_sem_logits, gumbel_noise, decodings)
